# baseline (speedup 1.0000x reference)
import jax
import jax.numpy as jnp
from jax import lax
from jax.experimental import pallas as pl
from jax.experimental.pallas import tpu as pltpu
from jax.experimental.pallas import tpu_sc as plsc

D = 64
SEQ = 200
BT = 32
BTW = 128
LANES = 16
K = 4

_info = plsc.get_sparse_core_info()
NC, NS = _info.num_cores, _info.num_subcores
NW = NC * NS


def _emb_body(ids_hbm, tok_hbm, pos_hbm, out_hbm, pos_v, idx_v, gin, gout,
              sem_g, sem_o):
    wid = lax.axis_index("s") * NC + lax.axis_index("c")
    units_per_w = (SEQ * BT) // NW
    first = wid * units_per_w
    n_outer = units_per_w // K
    iota16 = lax.iota(jnp.int32, LANES)

    pltpu.sync_copy(pos_hbm.at[pl.ds(0, SEQ)], pos_v)

    def unit_sbt(u):
        return u // BT, lax.rem(u, BT)

    def gather_start(b, u):
        s, bt = unit_sbt(u)
        pltpu.sync_copy(ids_hbm.at[pl.ds(s, 1), pl.ds(bt * BTW, BTW)],
                        idx_v.at[b])
        pltpu.async_copy(tok_hbm.at[idx_v.at[b].at[0]], gin.at[b], sem_g.at[b])

    def gather_wait(b, u):
        pltpu.make_async_copy(tok_hbm.at[idx_v.at[b].at[0]], gin.at[b],
                              sem_g.at[b]).wait()

    def store_start(b, u):
        s, bt = unit_sbt(u)
        for dt in range(D // 8):
            pltpu.async_copy(gout.at[b].at[pl.ds(8 * dt, 8)],
                             out_hbm.at[s, dt, bt], sem_o.at[b])

    def store_wait(b, u):
        s, bt = unit_sbt(u)
        for dt in range(D // 8):
            pltpu.make_async_copy(gout.at[b].at[pl.ds(8 * dt, 8)],
                                  out_hbm.at[s, dt, bt], sem_o.at[b]).wait()

    def compute(b, u):
        s, _ = unit_sbt(u)
        prow = [pos_v[s, pl.ds(t * LANES, LANES)] for t in range(D // LANES)]

        @plsc.parallel_loop(0, BTW, unroll=4)
        def row_body(r):
            for t in range(D // LANES):
                sl = pl.ds(t * LANES, LANES)
                gin[b, r, sl] = gin[b, r, sl] + prow[t]

        @plsc.parallel_loop(0, (BTW // LANES) * LANES, unroll=4)
        def bt_body(m):
            b0t = m // LANES
            k = lax.rem(m, LANES)
            brows = iota16 + b0t * LANES
            dcols0 = (iota16 + k) & (LANES - 1)
            for t in range(D // LANES):
                dcols = dcols0 + t * LANES
                vals = plsc.load_gather(gin.at[b], [brows, dcols])
                plsc.store_scatter(gout.at[b], [dcols, brows], vals)

    for b in range(K):
        gather_start(b, first + b)

    def outer(g, carry):
        for b in range(K):
            u = first + g * K + b
            gather_wait(b, u)

            @pl.when(g > 0)
            def _():
                store_wait(b, u - K)

            compute(b, u)
            store_start(b, u)

            @pl.when(g < n_outer - 1)
            def _():
                gather_start(b, u + K)
        return carry

    lax.fori_loop(0, n_outer, outer, 0)

    for b in range(K):
        store_wait(b, first + (n_outer - 1) * K + b)


def kernel(input_ids, tok_emb, pos_emb):
    B, S = input_ids.shape
    ids_t = input_ids.T.astype(jnp.int32)
    mesh = plsc.VectorSubcoreMesh(core_axis_name="c", subcore_axis_name="s")
    k = pl.kernel(
        _emb_body,
        mesh=mesh,
        out_type=jax.ShapeDtypeStruct((SEQ, D // 8, BT, 8, BTW), jnp.float32),
        scratch_types=[
            pltpu.VMEM((SEQ, D), jnp.float32),
            pltpu.VMEM((K, 1, BTW), jnp.int32),
            pltpu.VMEM((K, BTW, D), jnp.float32),
            pltpu.VMEM((K, D, BTW), jnp.float32),
            pltpu.SemaphoreType.DMA((K,)),
            pltpu.SemaphoreType.DMA((K,)),
        ],
        compiler_params=pltpu.CompilerParams(use_tc_tiling_on_sc=False,
                                             needs_layout_passes=False),
    )
    p = k(ids_t, tok_emb, pos_emb)
    return p.transpose(2, 4, 0, 1, 3).reshape(B, S, D)


# speedup vs baseline: 1.3962x; 1.3962x over previous
import jax
import jax.numpy as jnp
from jax import lax
from jax.experimental import pallas as pl
from jax.experimental.pallas import tpu as pltpu
from jax.experimental.pallas import tpu_sc as plsc

D = 64
SEQ = 200
BT = 32
BTW = 128
LANES = 16
K = 4

_info = plsc.get_sparse_core_info()
NC, NS = _info.num_cores, _info.num_subcores
NW = NC * NS


def _emb_body(ids_hbm, tok_hbm, pos_hbm, out_hbm, pos_v, idx_v, gin, gout,
              sem_g, sem_o):
    wid = lax.axis_index("s") * NC + lax.axis_index("c")
    units_per_w = (SEQ * BT) // NW
    first = wid * units_per_w
    n_outer = units_per_w // K
    iota16 = lax.iota(jnp.int32, LANES)

    pltpu.sync_copy(pos_hbm.at[pl.ds(0, SEQ)], pos_v)

    def unit_sbt(u):
        return u // BT, lax.rem(u, BT)

    pltpu.sync_copy(ids_hbm.at[pl.ds(first, units_per_w)], idx_v)

    def gather_start(b, u):
        pltpu.async_copy(tok_hbm.at[idx_v.at[u - first]], gin.at[b],
                         sem_g.at[b])

    def gather_wait(b, u):
        pltpu.make_async_copy(tok_hbm.at[idx_v.at[u - first]], gin.at[b],
                              sem_g.at[b]).wait()

    def store_start(b, u):
        s, bt = unit_sbt(u)
        for dt in range(D // 8):
            pltpu.async_copy(gout.at[b].at[pl.ds(8 * dt, 8)],
                             out_hbm.at[s, dt, bt], sem_o.at[b])

    def store_wait(b, u):
        s, bt = unit_sbt(u)
        for dt in range(D // 8):
            pltpu.make_async_copy(gout.at[b].at[pl.ds(8 * dt, 8)],
                                  out_hbm.at[s, dt, bt], sem_o.at[b]).wait()

    def compute(b, u):
        s, _ = unit_sbt(u)
        prow = [pos_v[s, pl.ds(t * LANES, LANES)] for t in range(D // LANES)]

        @plsc.parallel_loop(0, BTW, unroll=4)
        def row_body(r):
            for t in range(D // LANES):
                sl = pl.ds(t * LANES, LANES)
                gin[b, r, sl] = gin[b, r, sl] + prow[t]

        @plsc.parallel_loop(0, (BTW // LANES) * LANES, unroll=4)
        def bt_body(m):
            b0t = m // LANES
            k = lax.rem(m, LANES)
            brows = iota16 + b0t * LANES
            dcols0 = (iota16 + k) & (LANES - 1)
            for t in range(D // LANES):
                dcols = dcols0 + t * LANES
                vals = plsc.load_gather(gin.at[b], [brows, dcols])
                plsc.store_scatter(gout.at[b], [dcols, brows], vals)

    for b in range(K):
        gather_start(b, first + b)

    def outer(g, carry):
        for b in range(K):
            u = first + g * K + b
            gather_wait(b, u)

            @pl.when(g > 0)
            def _():
                store_wait(b, u - K)

            compute(b, u)
            store_start(b, u)

            @pl.when(g < n_outer - 1)
            def _():
                gather_start(b, u + K)
        return carry

    lax.fori_loop(0, n_outer, outer, 0)

    for b in range(K):
        store_wait(b, first + (n_outer - 1) * K + b)


def kernel(input_ids, tok_emb, pos_emb):
    B, S = input_ids.shape
    ids_t = input_ids.T.astype(jnp.int32).reshape(SEQ * BT, BTW)
    mesh = plsc.VectorSubcoreMesh(core_axis_name="c", subcore_axis_name="s")
    k = pl.kernel(
        _emb_body,
        mesh=mesh,
        out_type=jax.ShapeDtypeStruct((SEQ, D // 8, BT, 8, BTW), jnp.float32),
        scratch_types=[
            pltpu.VMEM((SEQ, D), jnp.float32),
            pltpu.VMEM((SEQ * BT // NW, BTW), jnp.int32),
            pltpu.VMEM((K, BTW, D), jnp.float32),
            pltpu.VMEM((K, D, BTW), jnp.float32),
            pltpu.SemaphoreType.DMA((K,)),
            pltpu.SemaphoreType.DMA((K,)),
        ],
        compiler_params=pltpu.CompilerParams(use_tc_tiling_on_sc=False,
                                             needs_layout_passes=False),
    )
    p = k(ids_t, tok_emb, pos_emb)
    return p.transpose(2, 4, 0, 1, 3).reshape(B, S, D)
